# baseline (speedup 1.0000x reference)
import functools
import jax
import jax.numpy as jnp
from jax import lax
from jax.experimental import pallas as pl
from jax.experimental.pallas import tpu as pltpu
from jax.experimental.pallas import tpu_sc as plsc

_CHUNK = 16


def _sc_blend(tab2, t, xt, nt):
  D, B = xt.shape
  T2 = tab2.shape[0]
  TP = T2 // 2
  info = plsc.get_sparse_core_info()
  nw = info.num_cores * info.num_subcores
  L = info.num_lanes
  rows = D // nw
  nch = rows // _CHUNK
  mesh = plsc.VectorSubcoreMesh(core_axis_name="c", subcore_axis_name="s")

  @functools.partial(
      pl.kernel,
      out_type=jax.ShapeDtypeStruct((D, B), jnp.float32),
      mesh=mesh,
      scratch_types=[
          pltpu.VMEM((T2,), jnp.float32),
          pltpu.VMEM((B,), jnp.int32),
          pltpu.VMEM((B,), jnp.float32),
          pltpu.VMEM((B,), jnp.float32),
          pltpu.VMEM((2, _CHUNK, B), jnp.float32),
          pltpu.VMEM((2, _CHUNK, B), jnp.float32),
          pltpu.VMEM((2, _CHUNK, B), jnp.float32),
          pltpu.SemaphoreType.DMA,
          pltpu.SemaphoreType.DMA,
          pltpu.SemaphoreType.DMA,
          pltpu.SemaphoreType.DMA,
          pltpu.SemaphoreType.DMA,
          pltpu.SemaphoreType.DMA,
      ],
      compiler_params=pltpu.CompilerParams(needs_layout_passes=False),
  )
  def blend_kernel(tab_hbm, t_hbm, x_hbm, n_hbm, o_hbm,
                   tab_v, idx_v, sa_v, s1a_v, xv, nv, ov,
                   sx0, sx1, sn0, sn1, so0, so1):
    sx = (sx0, sx1)
    sn = (sn0, sn1)
    so = (so0, so1)
    wid = lax.axis_index("s") * info.num_cores + lax.axis_index("c")
    rowbase = wid * rows

    cp_tab = pltpu.make_async_copy(tab_hbm, tab_v, sx0)
    cp_idx = pltpu.make_async_copy(t_hbm, idx_v, sn0)
    cp_tab.start()
    cp_idx.start()
    cp_tab.wait()
    cp_idx.wait()
    for j in range(B // L):
      idx = idx_v[pl.ds(j * L, L)]
      sa_v[pl.ds(j * L, L)] = plsc.load_gather(tab_v, [idx])
      s1a_v[pl.ds(j * L, L)] = plsc.load_gather(tab_v, [idx + TP])

    for b in range(2):
      r0 = rowbase + b * _CHUNK
      pltpu.make_async_copy(x_hbm.at[pl.ds(r0, _CHUNK)], xv.at[b], sx[b]).start()
      pltpu.make_async_copy(n_hbm.at[pl.ds(r0, _CHUNK)], nv.at[b], sn[b]).start()

    @pl.loop(0, nch, step=2)
    def _chunk_pair(c0):
      for b in range(2):
        c = c0 + b
        r0 = rowbase + c * _CHUNK
        pltpu.make_async_copy(x_hbm.at[pl.ds(r0, _CHUNK)], xv.at[b], sx[b]).wait()
        pltpu.make_async_copy(n_hbm.at[pl.ds(r0, _CHUNK)], nv.at[b], sn[b]).wait()

        @pl.when(c >= 2)
        def _drain_out():
          pltpu.make_async_copy(
              ov.at[b], o_hbm.at[pl.ds(r0, _CHUNK)], so[b]).wait()

        @pl.loop(0, B // L, unroll=2)
        def _lane_group(v):
          off = pl.multiple_of(v * L, L)
          sa16 = sa_v[pl.ds(off, L)]
          s1a16 = s1a_v[pl.ds(off, L)]
          for r in range(_CHUNK):
            ov[b, r, pl.ds(off, L)] = (
                sa16 * xv[b, r, pl.ds(off, L)]
                + s1a16 * nv[b, r, pl.ds(off, L)])

        pltpu.make_async_copy(
            ov.at[b], o_hbm.at[pl.ds(r0, _CHUNK)], so[b]).start()

        @pl.when(c + 2 < nch)
        def _next_in():
          r2 = r0 + 2 * _CHUNK
          pltpu.make_async_copy(
              x_hbm.at[pl.ds(r2, _CHUNK)], xv.at[b], sx[b]).start()
          pltpu.make_async_copy(
              n_hbm.at[pl.ds(r2, _CHUNK)], nv.at[b], sn[b]).start()

    for b in range(2):
      pltpu.make_async_copy(
          ov.at[b], o_hbm.at[pl.ds(rowbase, _CHUNK)], so[b]).wait()

  return blend_kernel(tab2, t, xt, nt)


@jax.jit
def kernel(x_start, noise, t, sqrt_alphas_cumprod, sqrt_one_minus_alphas_cumprod):
  B, C, H, W = x_start.shape
  D = C * H * W
  T = sqrt_alphas_cumprod.shape[0]
  TP = (T + 127) // 128 * 128
  tab2 = jnp.zeros((2 * TP,), jnp.float32)
  tab2 = lax.dynamic_update_slice(tab2, sqrt_alphas_cumprod, (0,))
  tab2 = lax.dynamic_update_slice(tab2, sqrt_one_minus_alphas_cumprod, (TP,))
  xt = jnp.transpose(x_start, (1, 2, 3, 0)).reshape(D, B)
  nt = jnp.transpose(noise, (1, 2, 3, 0)).reshape(D, B)
  out = _sc_blend(tab2, t, xt, nt)
  return jnp.transpose(out.reshape(C, H, W, B), (3, 0, 1, 2))


# speedup vs baseline: 1.6159x; 1.6159x over previous
import functools
import jax
import jax.numpy as jnp
from jax import lax
from jax.experimental import pallas as pl
from jax.experimental.pallas import tpu as pltpu
from jax.experimental.pallas import tpu_sc as plsc


def _sc_gather_coeffs(sa_tab, s1a_tab, t):
  B = t.shape[0]
  T = sa_tab.shape[0]
  TP = (T + 127) // 128 * 128
  T2 = 2 * TP
  info = plsc.get_sparse_core_info()
  nw = info.num_cores * info.num_subcores
  L = info.num_lanes
  b_per_w = B // nw
  mesh = plsc.VectorSubcoreMesh(core_axis_name="c", subcore_axis_name="s")

  @functools.partial(
      pl.kernel,
      out_type=(
          jax.ShapeDtypeStruct((B,), jnp.float32),
          jax.ShapeDtypeStruct((B,), jnp.float32),
      ),
      mesh=mesh,
      scratch_types=[
          pltpu.VMEM((T2,), jnp.float32),
          pltpu.VMEM((b_per_w,), jnp.int32),
          pltpu.VMEM((b_per_w,), jnp.float32),
          pltpu.VMEM((b_per_w,), jnp.float32),
          pltpu.SemaphoreType.DMA,
          pltpu.SemaphoreType.DMA,
          pltpu.SemaphoreType.DMA,
          pltpu.SemaphoreType.DMA,
      ],
      compiler_params=pltpu.CompilerParams(needs_layout_passes=False),
  )
  def gather_kernel(sa_hbm, s1a_hbm, t_hbm, osa_hbm, os1a_hbm,
                    tab_v, idx_v, osa_v, os1a_v, sem0, sem1, sem2, sem3):
    wid = lax.axis_index("s") * info.num_cores + lax.axis_index("c")
    base = wid * b_per_w
    cp_sa = pltpu.make_async_copy(sa_hbm, tab_v.at[pl.ds(0, T)], sem0)
    cp_s1a = pltpu.make_async_copy(s1a_hbm, tab_v.at[pl.ds(TP, T)], sem2)
    cp_idx = pltpu.make_async_copy(
        t_hbm.at[pl.ds(base, b_per_w)], idx_v, sem1)
    cp_sa.start()
    cp_s1a.start()
    cp_idx.start()
    cp_sa.wait()
    cp_s1a.wait()
    cp_idx.wait()
    for j in range(b_per_w // L):
      idx = idx_v[pl.ds(j * L, L)]
      osa_v[pl.ds(j * L, L)] = plsc.load_gather(tab_v, [idx])
      os1a_v[pl.ds(j * L, L)] = plsc.load_gather(tab_v, [idx + TP])
    cp_osa = pltpu.async_copy(osa_v, osa_hbm.at[pl.ds(base, b_per_w)], sem2)
    cp_os1a = pltpu.async_copy(os1a_v, os1a_hbm.at[pl.ds(base, b_per_w)], sem3)
    cp_osa.wait()
    cp_os1a.wait()

  return gather_kernel(sa_tab, s1a_tab, t)


def _blend_body(sa_ref, s1a_ref, x_ref, n_ref, o_ref):
  o_ref[...] = sa_ref[...] * x_ref[...] + s1a_ref[...] * n_ref[...]


def _tc_blend(sa, s1a, xt, nt, blk):
  D, B = xt.shape
  grid = (D // blk,)
  return pl.pallas_call(
      _blend_body,
      grid=grid,
      in_specs=[
          pl.BlockSpec((1, B), lambda i: (0, 0)),
          pl.BlockSpec((1, B), lambda i: (0, 0)),
          pl.BlockSpec((blk, B), lambda i: (i, 0)),
          pl.BlockSpec((blk, B), lambda i: (i, 0)),
      ],
      out_specs=pl.BlockSpec((blk, B), lambda i: (i, 0)),
      out_shape=jax.ShapeDtypeStruct((D, B), jnp.float32),
      compiler_params=pltpu.CompilerParams(
          dimension_semantics=("arbitrary",),
      ),
  )(sa, s1a, xt, nt)


@jax.jit
def kernel(x_start, noise, t, sqrt_alphas_cumprod, sqrt_one_minus_alphas_cumprod):
  B, C, H, W = x_start.shape
  D = C * H * W
  sa_g, s1a_g = _sc_gather_coeffs(
      sqrt_alphas_cumprod, sqrt_one_minus_alphas_cumprod, t)
  xt = jnp.transpose(x_start, (1, 2, 3, 0)).reshape(D, B)
  nt = jnp.transpose(noise, (1, 2, 3, 0)).reshape(D, B)
  out = _tc_blend(sa_g.reshape(1, B), s1a_g.reshape(1, B), xt, nt, blk=1024)
  return jnp.transpose(out.reshape(C, H, W, B), (3, 0, 1, 2))


# speedup vs baseline: 1.6349x; 1.0118x over previous
import functools
import jax
import jax.numpy as jnp
from jax import lax
from jax.experimental import pallas as pl
from jax.experimental.pallas import tpu as pltpu
from jax.experimental.pallas import tpu_sc as plsc


def _sc_gather_coeffs(tab2, t):
  B = t.shape[0]
  T2 = tab2.shape[0]
  TP = T2 // 2
  info = plsc.get_sparse_core_info()
  nw = info.num_cores * info.num_subcores
  L = info.num_lanes
  b_per_w = B // nw
  mesh = plsc.VectorSubcoreMesh(core_axis_name="c", subcore_axis_name="s")

  @functools.partial(
      pl.kernel,
      out_type=(
          jax.ShapeDtypeStruct((B,), jnp.float32),
          jax.ShapeDtypeStruct((B,), jnp.float32),
      ),
      mesh=mesh,
      scratch_types=[
          pltpu.VMEM((T2,), jnp.float32),
          pltpu.VMEM((b_per_w,), jnp.int32),
          pltpu.VMEM((b_per_w,), jnp.float32),
          pltpu.VMEM((b_per_w,), jnp.float32),
          pltpu.SemaphoreType.DMA,
          pltpu.SemaphoreType.DMA,
          pltpu.SemaphoreType.DMA,
          pltpu.SemaphoreType.DMA,
      ],
      compiler_params=pltpu.CompilerParams(needs_layout_passes=False),
  )
  def gather_kernel(tab_hbm, t_hbm, osa_hbm, os1a_hbm,
                    tab_v, idx_v, osa_v, os1a_v, sem0, sem1, sem2, sem3):
    wid = lax.axis_index("s") * info.num_cores + lax.axis_index("c")
    base = wid * b_per_w
    cp_tab = pltpu.async_copy(tab_hbm, tab_v, sem0)
    cp_idx = pltpu.async_copy(t_hbm.at[pl.ds(base, b_per_w)], idx_v, sem1)
    cp_tab.wait()
    cp_idx.wait()
    for j in range(b_per_w // L):
      idx = idx_v[pl.ds(j * L, L)]
      osa_v[pl.ds(j * L, L)] = plsc.load_gather(tab_v, [idx])
      os1a_v[pl.ds(j * L, L)] = plsc.load_gather(tab_v, [idx + TP])
    cp_osa = pltpu.async_copy(osa_v, osa_hbm.at[pl.ds(base, b_per_w)], sem2)
    cp_os1a = pltpu.async_copy(os1a_v, os1a_hbm.at[pl.ds(base, b_per_w)], sem3)
    cp_osa.wait()
    cp_os1a.wait()

  return gather_kernel(tab2, t)


def _blend_body(sa_ref, s1a_ref, x_ref, n_ref, o_ref):
  o_ref[...] = sa_ref[...] * x_ref[...] + s1a_ref[...] * n_ref[...]


def _tc_blend(sa, s1a, xt, nt, blk):
  D, B = xt.shape
  grid = (D // blk,)
  return pl.pallas_call(
      _blend_body,
      grid=grid,
      in_specs=[
          pl.BlockSpec((1, B), lambda i: (0, 0)),
          pl.BlockSpec((1, B), lambda i: (0, 0)),
          pl.BlockSpec((blk, B), lambda i: (i, 0)),
          pl.BlockSpec((blk, B), lambda i: (i, 0)),
      ],
      out_specs=pl.BlockSpec((blk, B), lambda i: (i, 0)),
      out_shape=jax.ShapeDtypeStruct((D, B), jnp.float32),
      compiler_params=pltpu.CompilerParams(
          dimension_semantics=("arbitrary",),
      ),
  )(sa, s1a, xt, nt)


@jax.jit
def kernel(x_start, noise, t, sqrt_alphas_cumprod, sqrt_one_minus_alphas_cumprod):
  B, C, H, W = x_start.shape
  D = C * H * W
  T = sqrt_alphas_cumprod.shape[0]
  TP = (T + 127) // 128 * 128
  tab2 = jnp.zeros((2 * TP,), jnp.float32)
  tab2 = lax.dynamic_update_slice(tab2, sqrt_alphas_cumprod, (0,))
  tab2 = lax.dynamic_update_slice(tab2, sqrt_one_minus_alphas_cumprod, (TP,))
  sa_g, s1a_g = _sc_gather_coeffs(tab2, t)
  xt = jnp.transpose(x_start, (1, 2, 3, 0)).reshape(D, B)
  nt = jnp.transpose(noise, (1, 2, 3, 0)).reshape(D, B)
  out = _tc_blend(sa_g.reshape(1, B), s1a_g.reshape(1, B), xt, nt, blk=1024)
  return jnp.transpose(out.reshape(C, H, W, B), (3, 0, 1, 2))


# speedup vs baseline: 1.6382x; 1.0020x over previous
import functools
import jax
import jax.numpy as jnp
from jax import lax
from jax.experimental import pallas as pl
from jax.experimental.pallas import tpu as pltpu
from jax.experimental.pallas import tpu_sc as plsc


def _sc_gather_coeffs(tab2, t):
  B = t.shape[0]
  T2 = tab2.shape[0]
  TP = T2 // 2
  info = plsc.get_sparse_core_info()
  nw = info.num_cores * info.num_subcores
  L = info.num_lanes
  b_per_w = B // nw
  mesh = plsc.VectorSubcoreMesh(core_axis_name="c", subcore_axis_name="s")

  @functools.partial(
      pl.kernel,
      out_type=(
          jax.ShapeDtypeStruct((B,), jnp.float32),
          jax.ShapeDtypeStruct((B,), jnp.float32),
      ),
      mesh=mesh,
      scratch_types=[
          pltpu.VMEM((T2,), jnp.float32),
          pltpu.VMEM((b_per_w,), jnp.int32),
          pltpu.VMEM((b_per_w,), jnp.float32),
          pltpu.VMEM((b_per_w,), jnp.float32),
          pltpu.SemaphoreType.DMA,
          pltpu.SemaphoreType.DMA,
          pltpu.SemaphoreType.DMA,
          pltpu.SemaphoreType.DMA,
      ],
      compiler_params=pltpu.CompilerParams(needs_layout_passes=False, skip_device_barrier=True),
  )
  def gather_kernel(tab_hbm, t_hbm, osa_hbm, os1a_hbm,
                    tab_v, idx_v, osa_v, os1a_v, sem0, sem1, sem2, sem3):
    wid = lax.axis_index("s") * info.num_cores + lax.axis_index("c")
    base = wid * b_per_w
    cp_tab = pltpu.async_copy(tab_hbm, tab_v, sem0)
    cp_idx = pltpu.async_copy(t_hbm.at[pl.ds(base, b_per_w)], idx_v, sem1)
    cp_tab.wait()
    cp_idx.wait()
    for j in range(b_per_w // L):
      idx = idx_v[pl.ds(j * L, L)]
      osa_v[pl.ds(j * L, L)] = plsc.load_gather(tab_v, [idx])
      os1a_v[pl.ds(j * L, L)] = plsc.load_gather(tab_v, [idx + TP])
    cp_osa = pltpu.async_copy(osa_v, osa_hbm.at[pl.ds(base, b_per_w)], sem2)
    cp_os1a = pltpu.async_copy(os1a_v, os1a_hbm.at[pl.ds(base, b_per_w)], sem3)
    cp_osa.wait()
    cp_os1a.wait()

  return gather_kernel(tab2, t)


def _blend_body(sa_ref, s1a_ref, x_ref, n_ref, o_ref):
  o_ref[...] = sa_ref[...] * x_ref[...] + s1a_ref[...] * n_ref[...]


def _tc_blend(sa, s1a, xt, nt, blk):
  D, B = xt.shape
  grid = (D // blk,)
  return pl.pallas_call(
      _blend_body,
      grid=grid,
      in_specs=[
          pl.BlockSpec((1, B), lambda i: (0, 0)),
          pl.BlockSpec((1, B), lambda i: (0, 0)),
          pl.BlockSpec((blk, B), lambda i: (i, 0)),
          pl.BlockSpec((blk, B), lambda i: (i, 0)),
      ],
      out_specs=pl.BlockSpec((blk, B), lambda i: (i, 0)),
      out_shape=jax.ShapeDtypeStruct((D, B), jnp.float32),
      compiler_params=pltpu.CompilerParams(
          dimension_semantics=("arbitrary",),
          skip_device_barrier=True,
      ),
  )(sa, s1a, xt, nt)


@jax.jit
def kernel(x_start, noise, t, sqrt_alphas_cumprod, sqrt_one_minus_alphas_cumprod):
  B, C, H, W = x_start.shape
  D = C * H * W
  T = sqrt_alphas_cumprod.shape[0]
  TP = (T + 127) // 128 * 128
  tab2 = jnp.zeros((2 * TP,), jnp.float32)
  tab2 = lax.dynamic_update_slice(tab2, sqrt_alphas_cumprod, (0,))
  tab2 = lax.dynamic_update_slice(tab2, sqrt_one_minus_alphas_cumprod, (TP,))
  sa_g, s1a_g = _sc_gather_coeffs(tab2, t)
  xt = jnp.transpose(x_start, (1, 2, 3, 0)).reshape(D, B)
  nt = jnp.transpose(noise, (1, 2, 3, 0)).reshape(D, B)
  out = _tc_blend(sa_g.reshape(1, B), s1a_g.reshape(1, B), xt, nt, blk=1024)
  return jnp.transpose(out.reshape(C, H, W, B), (3, 0, 1, 2))
